# baseline (speedup 1.0000x reference)
import functools

import jax
import jax.numpy as jnp
from jax import lax
from jax.experimental import pallas as pl
from jax.experimental.pallas import tpu as pltpu
from jax.experimental.pallas import tpu_sc as plsc

G = 100000
B = 16384
D0, D1, D2 = 16, 64, 256
OUT_D = D0 + D1 + D2
DP = 384

_info = plsc.get_sparse_core_info()
NC, NS = _info.num_cores, _info.num_subcores
NW = NC * NS
BPW = B // NW
CH = 128
NCH = BPW // CH

_mesh = plsc.VectorSubcoreMesh(core_axis_name="c", subcore_axis_name="s")


@functools.partial(
    pl.kernel,
    mesh=_mesh,
    out_type=jax.ShapeDtypeStruct((B, DP), jnp.float32),
    compiler_params=pltpu.CompilerParams(use_tc_tiling_on_sc=True),
    scratch_types=[
        pltpu.VMEM((NCH, CH), jnp.int32),
        pltpu.VMEM((2 * CH, DP), jnp.float32),
        pltpu.SemaphoreType.DMA,
    ],
)
def _sc_gather(x_hbm, tab_hbm, out_hbm, idx_v, rows_v, sem):
    wid = lax.axis_index("s") * NC + lax.axis_index("c")
    base = wid * BPW

    pltpu.sync_copy(x_hbm.at[pl.ds(wid * NCH, NCH)], idx_v)

    def fire(j):
        return pltpu.async_copy(tab_hbm.at[idx_v.at[j]],
                                rows_v.at[pl.ds((j % 2) * CH, CH)], sem)

    h = fire(0)
    hn = fire(1)
    for j in range(NCH):
        h.wait()
        pltpu.sync_copy(rows_v.at[pl.ds((j % 2) * CH, CH)],
                        out_hbm.at[pl.ds(base + j * CH, CH)])
        h = hn
        if j + 2 < NCH:
            hn = fire(j + 2)


def kernel(x, rep0, rep1, rep2):
    x2 = x.astype(jnp.int32).reshape(B // CH, CH)
    tab = jnp.concatenate(
        [rep0.reshape(G, D0), rep1.reshape(G, D1), rep2.reshape(G, D2),
         jnp.zeros((G, DP - OUT_D), jnp.float32)], axis=1)
    return _sc_gather(x2, tab)[:, :OUT_D]
